# baseline (speedup 1.0000x reference)
import functools

import jax
import jax.numpy as jnp
from jax import lax
from jax.experimental import pallas as pl
from jax.experimental.pallas import tpu as pltpu
from jax.experimental.pallas import tpu_sc as plsc

_BATCH = 16384
_HIST = 50
_D = 64
_VOCAB = 1000000
_NC = 2
_NS = 16
_NW = _NC * _NS
_BW = _BATCH // _NW
_C = 128
_PER_J = _BW // _C
_NBUF = 4
_NV = _HIST * _PER_J
_KB = _C // 16


def _make_emb_kernel():
  mesh = plsc.VectorSubcoreMesh(core_axis_name="c", subcore_axis_name="s")

  @functools.partial(
      pl.kernel,
      mesh=mesh,
      compiler_params=pltpu.CompilerParams(needs_layout_passes=False),
      out_type=jax.ShapeDtypeStruct((_HIST, _D, _BATCH), jnp.float32),
      scratch_types=(
          [pltpu.VMEM((_HIST, _BW), jnp.int32)]
          + [pltpu.VMEM((_C,), jnp.int32) for _ in range(_NBUF)]
          + [pltpu.VMEM((_C, 2 * _D), jnp.float32) for _ in range(_NBUF)]
          + [pltpu.VMEM((_D, _C), jnp.float32) for _ in range(_NBUF)]
          + [pltpu.SemaphoreType.DMA for _ in range(2 * _NBUF)]
      ),
  )
  def emb(xt_hbm, wt_hbm, ot_hbm, idx_v, *bufs):
    idxg = bufs[:_NBUF]
    rows = bufs[_NBUF:2 * _NBUF]
    tbuf = bufs[2 * _NBUF:3 * _NBUF]
    gsem = bufs[3 * _NBUF:4 * _NBUF]
    ssem = bufs[4 * _NBUF:]
    wid = lax.axis_index("s") * _NC + lax.axis_index("c")
    base = wid * _BW
    iota = lax.iota(jnp.int32, 16)

    pltpu.sync_copy(xt_hbm.at[:, pl.ds(base, _BW)], idx_v)

    def fire(v, b):
      j = v // _PER_J
      s = lax.rem(v, _PER_J)
      col0 = s * _C
      def prep(k, carry):
        vec = idx_v[j, pl.ds(col0 + k * 16, 16)]
        idxg[b][pl.ds(k * 16, 16)] = lax.shift_right_logical(vec, 1)
        return carry
      lax.fori_loop(0, _KB, prep, 0)
      pltpu.async_copy(wt_hbm.at[idxg[b]], rows[b], gsem[b])

    def drain_g(b):
      pltpu.make_async_copy(wt_hbm.at[pl.ds(0, _C)], rows[b], gsem[b]).wait()

    def drain_s(b):
      pltpu.make_async_copy(
          ot_hbm.at[0, :, pl.ds(0, _C)], tbuf[b], ssem[b]
      ).wait()

    def transpose(v, b):
      j = v // _PER_J
      s = lax.rem(v, _PER_J)
      col0 = s * _C
      def blk(k, carry):
        vec = idx_v[j, pl.ds(col0 + k * 16, 16)]
        half = lax.shift_left(lax.bitwise_and(vec, 1), 6)
        rowsidx = iota + k * 16
        for f in range(_D):
          val = plsc.load_gather(rows[b], [rowsidx, half + f])
          tbuf[b][f, pl.ds(k * 16, 16)] = val
        return carry
      lax.fori_loop(0, _KB, blk, 0)

    def store(v, b):
      j = v // _PER_J
      s = lax.rem(v, _PER_J)
      pltpu.async_copy(
          tbuf[b],
          ot_hbm.at[j, :, pl.ds(base + s * _C, _C)],
          ssem[b],
      )

    for b in range(_NBUF):
      fire(b, b)

    for v in range(_NBUF):
      b = v % _NBUF
      drain_g(b)
      transpose(v, b)
      fire(v + _NBUF, b)
      store(v, b)

    def body(h, carry):
      for b in range(_NBUF):
        v = _NBUF + h * _NBUF + b
        drain_g(b)
        drain_s(b)
        transpose(v, b)
        fire(v + _NBUF, b)
        store(v, b)
      return carry

    lax.fori_loop(0, (_NV - 2 * _NBUF) // _NBUF, body, 0)

    for v in range(_NV - _NBUF, _NV):
      b = v % _NBUF
      drain_g(b)
      drain_s(b)
      transpose(v, b)
      store(v, b)

    for b in range(_NBUF):
      drain_s(b)

  return emb


_EMB = _make_emb_kernel()


@jax.jit
def kernel(x, lut_weight):
  xt = jnp.transpose(x)
  wt = jnp.reshape(lut_weight, (_VOCAB // 2, 2 * _D))
  ot = _EMB(xt, wt)
  return jnp.transpose(ot, (2, 0, 1))


# speedup vs baseline: 1.4713x; 1.4713x over previous
import functools

import jax
import jax.numpy as jnp
from jax import lax
from jax.experimental import pallas as pl
from jax.experimental.pallas import tpu as pltpu
from jax.experimental.pallas import tpu_sc as plsc

_BATCH = 16384
_HIST = 50
_D = 64
_B = _BATCH * _HIST
_NC = 2
_NS = 16
_NW = _NC * _NS
_B_PER_W = _B // _NW
_GROUP = 512
_NBUF = 3
_N_GROUPS = _B_PER_W // _GROUP


def _make_emb_kernel():
  mesh = plsc.VectorSubcoreMesh(core_axis_name="c", subcore_axis_name="s")

  @functools.partial(
      pl.kernel,
      mesh=mesh,
      compiler_params=pltpu.CompilerParams(use_tc_tiling_on_sc=False),
      out_type=jax.ShapeDtypeStruct((_B, _D), jnp.float32),
      scratch_types=(
          [pltpu.VMEM((_B_PER_W,), jnp.int32)]
          + [pltpu.VMEM((_GROUP, _D), jnp.float32) for _ in range(_NBUF)]
          + [pltpu.SemaphoreType.DMA for _ in range(_NBUF)]
      ),
  )
  def emb(idx_hbm, table_hbm, out_hbm, idx_v, *bufs):
    rows = bufs[:_NBUF]
    gsem = bufs[_NBUF:]
    wid = lax.axis_index("s") * _NC + lax.axis_index("c")
    base = wid * _B_PER_W

    pltpu.sync_copy(idx_hbm.at[pl.ds(base, _B_PER_W)], idx_v)

    def fire(g, b):
      pltpu.async_copy(
          table_hbm.at[idx_v.at[pl.ds(g * _GROUP, _GROUP)]],
          rows[b],
          gsem[b],
      )

    def drain(b):
      pltpu.make_async_copy(
          out_hbm.at[pl.ds(0, _GROUP)],
          rows[b],
          gsem[b],
      ).wait()

    def store(g, b):
      pltpu.sync_copy(rows[b], out_hbm.at[pl.ds(base + g * _GROUP, _GROUP)])

    for b in range(_NBUF):
      fire(b, b)

    n_main = (_N_GROUPS - _NBUF) // _NBUF * _NBUF

    def body(h, carry):
      for b in range(_NBUF):
        g = h * _NBUF + b
        drain(b)
        store(g, b)
        fire(g + _NBUF, b)
      return carry

    lax.fori_loop(0, n_main // _NBUF, body, 0)

    for g in range(n_main, _N_GROUPS - _NBUF):
      b = g % _NBUF
      drain(b)
      store(g, b)
      fire(g + _NBUF, b)

    for g in range(_N_GROUPS - _NBUF, _N_GROUPS):
      b = g % _NBUF
      drain(b)
      store(g, b)

  return emb


_EMB = _make_emb_kernel()


@jax.jit
def kernel(x, lut_weight):
  idx = x.reshape(_B).astype(jnp.int32)
  out = _EMB(idx, lut_weight)
  return out.reshape(_BATCH, _HIST, _D)
